# Initial kernel scaffold; baseline (speedup 1.0000x reference)
import jax
import jax.numpy as jnp
from jax.experimental import pallas as pl


def kernel(img, w1, b1, w2, b2, w3, b3, w4, b4, ws, bs, wc, bc, wd, bd):
    raise NotImplementedError("write your pallas kernel here")


# speedup vs baseline: 1.1972x; 1.1972x over previous
import functools

import jax
import jax.numpy as jnp
from jax import lax
from jax.experimental import pallas as pl
from jax.experimental.pallas import tpu as pltpu


def _halo(w):
    return ((w + 1 + 7) // 8) * 8


def _conv3x3(hsc, isc, w_ref, b_ref, x, Wl, HWl, P, pad_val):
    cin = x.shape[1]
    fill = jnp.full((P, cin), pad_val, dtype=jnp.float32)
    hsc[0:P, :] = fill
    hsc[P:P + HWl, :] = x
    hsc[P + HWl:P + HWl + P, :] = fill

    jcol = lax.broadcasted_iota(jnp.int32, (HWl, 1), 0) % Wl
    left_ok = jcol > 0
    right_ok = jcol < Wl - 1

    for di in range(3):
        for dj in range(3):
            t = di * 3 + dj
            off = P + (di - 1) * Wl + (dj - 1)
            v = hsc[pl.ds(off, HWl), :]
            if dj == 0:
                v = jnp.where(left_ok, v, pad_val)
            elif dj == 2:
                v = jnp.where(right_ok, v, pad_val)
            isc[:, t * cin:(t + 1) * cin] = v
    return jnp.dot(isc[...], w_ref[...],
                   preferred_element_type=jnp.float32) + b_ref[...]


def _point_net_kernel(
    x_ref, w1_ref, b1_ref, w2_ref, b2_ref, w3_ref, b3_ref, w4_ref, b4_ref,
    wh_ref, bh_ref,
    out_ref,
    hs1, ic1, hs2, ic2, ds2, hs3, ic3, hs4, ic4, ds4, hsh, ich,
    *, H, W, n_valid,
):
    HW = H * W
    H2, W2 = H // 2, W // 2
    HW2 = H2 * W2
    hc, wc = H // 4, W // 4
    HWc = hc * wc
    P1, P2, P3 = _halo(W), _halo(W2), _halo(wc)
    c2 = w2_ref.shape[1]
    c4 = w4_ref.shape[1]

    x0 = x_ref[...]
    a1 = jnp.maximum(_conv3x3(hs1, ic1, w1_ref, b1_ref, x0, W, HW, P1, 0.5), 0.0)
    a2 = jnp.maximum(_conv3x3(hs2, ic2, w2_ref, b2_ref, a1, W, HW, P1, 0.0), 0.0)

    ds2[...] = a2.reshape(H2, 2 * W, c2)
    a2d = ds2[:, pl.ds(0, W2, 2), :].reshape(HW2, c2)

    a3 = jnp.maximum(_conv3x3(hs3, ic3, w3_ref, b3_ref, a2d, W2, HW2, P2, 0.0), 0.0)
    a4 = jnp.maximum(_conv3x3(hs4, ic4, w4_ref, b4_ref, a3, W2, HW2, P2, 0.0), 0.0)

    ds4[...] = a4.reshape(H2 // 2, 2 * W2, c4)
    feat = ds4[:, pl.ds(0, wc, 2), :].reshape(HWc, c4)

    h = _conv3x3(hsh, ich, wh_ref, bh_ref, feat, wc, HWc, P3, 0.0)

    col = lax.broadcasted_iota(jnp.int32, h.shape, 1)
    cell = H // hc
    step = (cell - 1) / 2.0
    k = lax.broadcasted_iota(jnp.int32, (HWc, 1), 0)
    bx = (k % wc).astype(jnp.float32) * cell + step
    by = (k // wc).astype(jnp.float32) * cell + step
    base = jnp.where(col == 1, bx, by)
    cmax = jnp.where(col == 1, float(W - 1), float(H - 1))
    coord = jnp.clip(base + jnp.tanh(h) * step, 0.0, cmax)
    score = jax.nn.sigmoid(h)
    out = jnp.where(col == 0, score, jnp.where(col <= 2, coord, h))
    out_ref[...] = jnp.where(col < n_valid, out, 0.0)


def kernel(img, w1, b1, w2, b2, w3, b3, w4, b4, ws, bs, wc, bc, wd, bd):
    B, C, H, W = img.shape
    hc, wcell = H // 4, W // 4
    HW, HW2, HWc = H * W, (H // 2) * (W // 2), hc * wcell
    c1 = w1.shape[1]
    c2 = w2.shape[1]
    c3 = w3.shape[1]
    c4 = w4.shape[1]
    c_desc = wd.shape[1]
    n_valid = 3 + c_desc
    NH = 128

    x = jnp.transpose(img, (0, 2, 3, 1)).reshape(B * HW, C)

    inv_std = 1.0 / 0.225
    w1f = w1 * inv_std
    b1f = b1 - (0.5 * inv_std) * jnp.sum(w1, axis=0, keepdims=True)

    wh = jnp.concatenate([ws, wc, wd], axis=1)
    bh = jnp.concatenate([bs, bc, bd], axis=1)
    wh = jnp.pad(wh, ((0, 0), (0, NH - n_valid)))
    bh = jnp.pad(bh, ((0, 0), (0, NH - n_valid)))

    P1, P2, P3 = _halo(W), _halo(W // 2), _halo(wcell)
    full = lambda a: pl.BlockSpec(a.shape, lambda b: (0, 0))
    f32 = jnp.float32

    out = pl.pallas_call(
        functools.partial(_point_net_kernel, H=H, W=W, n_valid=n_valid),
        out_shape=jax.ShapeDtypeStruct((B * HWc, NH), f32),
        grid=(B,),
        in_specs=[
            pl.BlockSpec((HW, C), lambda b: (b, 0)),
            full(w1f), full(b1f),
            full(w2), full(b2),
            full(w3), full(b3),
            full(w4), full(b4),
            full(wh), full(bh),
        ],
        out_specs=pl.BlockSpec((HWc, NH), lambda b: (b, 0)),
        scratch_shapes=[
            pltpu.VMEM((HW + 2 * P1, C), f32),
            pltpu.VMEM((HW, 9 * C), f32),
            pltpu.VMEM((HW + 2 * P1, c1), f32),
            pltpu.VMEM((HW, 9 * c1), f32),
            pltpu.VMEM((H // 2, 2 * W, c2), f32),
            pltpu.VMEM((HW2 + 2 * P2, c2), f32),
            pltpu.VMEM((HW2, 9 * c2), f32),
            pltpu.VMEM((HW2 + 2 * P2, c3), f32),
            pltpu.VMEM((HW2, 9 * c3), f32),
            pltpu.VMEM((H // 4, 2 * (W // 2), c4), f32),
            pltpu.VMEM((HWc + 2 * P3, c4), f32),
            pltpu.VMEM((HWc, 9 * c4), f32),
        ],
        compiler_params=pltpu.CompilerParams(
            dimension_semantics=("parallel",),
            vmem_limit_bytes=64 * 1024 * 1024,
        ),
    )(x, w1f, b1f, w2, b2, w3, b3, w4, b4, wh, bh)

    out = out.reshape(B, hc, wcell, NH)
    score = jnp.transpose(out[..., 0:1], (0, 3, 1, 2))
    coord = jnp.transpose(out[..., 1:3], (0, 3, 1, 2))
    desc = jnp.transpose(out[..., 3:3 + c_desc], (0, 3, 1, 2))
    return score, coord, desc


# speedup vs baseline: 2.5729x; 2.1490x over previous
import functools

import jax
import jax.numpy as jnp
from jax import lax
from jax.experimental import pallas as pl
from jax.experimental.pallas import tpu as pltpu


def _halo(w):
    return ((w + 1 + 7) // 8) * 8


def _conv3x3_cm_in(hsc, isc, w_ref, b_ref, x, Wl, HWl, P, pad_val):
    cin = x.shape[0]
    fill = jnp.full((cin, P), pad_val, dtype=jnp.float32)
    hsc[:, 0:P] = fill
    hsc[:, P:P + HWl] = x
    hsc[:, P + HWl:P + HWl + P] = fill

    jcol = lax.broadcasted_iota(jnp.int32, (1, HWl), 1) % Wl
    left_ok = jcol > 0
    right_ok = jcol < Wl - 1

    for di in range(3):
        for dj in range(3):
            t = di * 3 + dj
            off = P + (di - 1) * Wl + (dj - 1)
            v = hsc[:, pl.ds(off, HWl)]
            if dj == 0:
                v = jnp.where(left_ok, v, pad_val)
            elif dj == 2:
                v = jnp.where(right_ok, v, pad_val)
            isc[t * cin:(t + 1) * cin, :] = v
    return lax.dot_general(
        isc[...], w_ref[...], (((0,), (0,)), ((), ())),
        preferred_element_type=jnp.float32) + b_ref[...]


def _conv3x3(hsc, isc, w_ref, b_ref, x, Wl, HWl, P, pad_val):
    cin = x.shape[1]
    fill = jnp.full((P, cin), pad_val, dtype=jnp.float32)
    hsc[0:P, :] = fill
    hsc[P:P + HWl, :] = x
    hsc[P + HWl:P + HWl + P, :] = fill

    jcol = lax.broadcasted_iota(jnp.int32, (HWl, 1), 0) % Wl
    left_ok = jcol > 0
    right_ok = jcol < Wl - 1

    for di in range(3):
        for dj in range(3):
            t = di * 3 + dj
            off = P + (di - 1) * Wl + (dj - 1)
            v = hsc[pl.ds(off, HWl), :]
            if dj == 0:
                v = jnp.where(left_ok, v, pad_val)
            elif dj == 2:
                v = jnp.where(right_ok, v, pad_val)
            isc[:, t * cin:(t + 1) * cin] = v
    return jnp.dot(isc[...], w_ref[...],
                   preferred_element_type=jnp.float32) + b_ref[...]


def _point_net_kernel(
    x_ref, w1_ref, b1_ref, w2_ref, b2_ref, w3_ref, b3_ref, w4_ref, b4_ref,
    wh_ref, bh_ref,
    out_ref,
    hs1, ic1, hs2, ic2, ds2, hs3, ic3, hs4, ic4, ds4, hsh, ich,
    *, H, W, n_valid,
):
    HW = H * W
    H2, W2 = H // 2, W // 2
    HW2 = H2 * W2
    hc, wc = H // 4, W // 4
    HWc = hc * wc
    P1, P2, P3 = _halo(W), _halo(W2), _halo(wc)
    c2 = w2_ref.shape[1]
    c4 = w4_ref.shape[1]

    x0 = x_ref[0]
    a1 = jnp.maximum(
        _conv3x3_cm_in(hs1, ic1, w1_ref, b1_ref, x0, W, HW, P1, 0.5), 0.0)
    a2 = jnp.maximum(_conv3x3(hs2, ic2, w2_ref, b2_ref, a1, W, HW, P1, 0.0), 0.0)

    ds2[...] = a2.reshape(H2, 2 * W, c2)
    a2d = ds2[:, pl.ds(0, W2, 2), :].reshape(HW2, c2)

    a3 = jnp.maximum(_conv3x3(hs3, ic3, w3_ref, b3_ref, a2d, W2, HW2, P2, 0.0), 0.0)
    a4 = jnp.maximum(_conv3x3(hs4, ic4, w4_ref, b4_ref, a3, W2, HW2, P2, 0.0), 0.0)

    ds4[...] = a4.reshape(H2 // 2, 2 * W2, c4)
    feat = ds4[:, pl.ds(0, wc, 2), :].reshape(HWc, c4)

    h = _conv3x3(hsh, ich, wh_ref, bh_ref, feat, wc, HWc, P3, 0.0)

    col = lax.broadcasted_iota(jnp.int32, h.shape, 1)
    cell = H // hc
    step = (cell - 1) / 2.0
    k = lax.broadcasted_iota(jnp.int32, (HWc, 1), 0)
    bx = (k % wc).astype(jnp.float32) * cell + step
    by = (k // wc).astype(jnp.float32) * cell + step
    base = jnp.where(col == 1, bx, by)
    cmax = jnp.where(col == 1, float(W - 1), float(H - 1))
    coord = jnp.clip(base + jnp.tanh(h) * step, 0.0, cmax)
    score = jax.nn.sigmoid(h)
    out = jnp.where(col == 0, score, jnp.where(col <= 2, coord, h))
    out = jnp.where(col < n_valid, out, 0.0)
    out_ref[...] = out.T


def kernel(img, w1, b1, w2, b2, w3, b3, w4, b4, ws, bs, wc, bc, wd, bd):
    B, C, H, W = img.shape
    hc, wcell = H // 4, W // 4
    HW, HW2, HWc = H * W, (H // 2) * (W // 2), hc * wcell
    c1 = w1.shape[1]
    c2 = w2.shape[1]
    c3 = w3.shape[1]
    c4 = w4.shape[1]
    c_desc = wd.shape[1]
    n_valid = 3 + c_desc
    NH = 128

    x = img.reshape(B, C, HW)

    inv_std = 1.0 / 0.225
    w1f = w1 * inv_std
    b1f = b1 - (0.5 * inv_std) * jnp.sum(w1, axis=0, keepdims=True)

    wh = jnp.concatenate([ws, wc, wd], axis=1)
    bh = jnp.concatenate([bs, bc, bd], axis=1)
    wh = jnp.pad(wh, ((0, 0), (0, NH - n_valid)))
    bh = jnp.pad(bh, ((0, 0), (0, NH - n_valid)))

    P1, P2, P3 = _halo(W), _halo(W // 2), _halo(wcell)
    full = lambda a: pl.BlockSpec(a.shape, lambda b: (0, 0))
    f32 = jnp.float32

    out = pl.pallas_call(
        functools.partial(_point_net_kernel, H=H, W=W, n_valid=n_valid),
        out_shape=jax.ShapeDtypeStruct((B * NH, HWc), f32),
        grid=(B,),
        in_specs=[
            pl.BlockSpec((1, C, HW), lambda b: (b, 0, 0)),
            full(w1f), full(b1f),
            full(w2), full(b2),
            full(w3), full(b3),
            full(w4), full(b4),
            full(wh), full(bh),
        ],
        out_specs=pl.BlockSpec((NH, HWc), lambda b: (b, 0)),
        scratch_shapes=[
            pltpu.VMEM((C, HW + 2 * P1), f32),
            pltpu.VMEM((9 * C, HW), f32),
            pltpu.VMEM((HW + 2 * P1, c1), f32),
            pltpu.VMEM((HW, 9 * c1), f32),
            pltpu.VMEM((H // 2, 2 * W, c2), f32),
            pltpu.VMEM((HW2 + 2 * P2, c2), f32),
            pltpu.VMEM((HW2, 9 * c2), f32),
            pltpu.VMEM((HW2 + 2 * P2, c3), f32),
            pltpu.VMEM((HW2, 9 * c3), f32),
            pltpu.VMEM((H // 4, 2 * (W // 2), c4), f32),
            pltpu.VMEM((HWc + 2 * P3, c4), f32),
            pltpu.VMEM((HWc, 9 * c4), f32),
        ],
        compiler_params=pltpu.CompilerParams(
            dimension_semantics=("parallel",),
            vmem_limit_bytes=64 * 1024 * 1024,
        ),
    )(x, w1f, b1f, w2, b2, w3, b3, w4, b4, wh, bh)

    out = out.reshape(B, NH, hc, wcell)
    score = out[:, 0:1]
    coord = out[:, 1:3]
    desc = out[:, 3:3 + c_desc]
    return score, coord, desc


# speedup vs baseline: 2.7290x; 1.0607x over previous
import functools

import jax
import jax.numpy as jnp
from jax import lax
from jax.experimental import pallas as pl
from jax.experimental.pallas import tpu as pltpu


def _halo(w):
    return ((w + 1 + 7) // 8) * 8


def _conv3x3_cm_in(hsc, isc, w_ref, b_ref, x, Wl, HWl, P, pad_val):
    cin = x.shape[0]
    fill = jnp.full((cin, P), pad_val, dtype=jnp.float32)
    hsc[:, 0:P] = fill
    hsc[:, P:P + HWl] = x
    hsc[:, P + HWl:P + HWl + P] = fill

    jcol = lax.broadcasted_iota(jnp.int32, (1, HWl), 1) % Wl
    left_ok = jcol > 0
    right_ok = jcol < Wl - 1

    for di in range(3):
        for dj in range(3):
            t = di * 3 + dj
            off = P + (di - 1) * Wl + (dj - 1)
            v = hsc[:, pl.ds(off, HWl)]
            if dj == 0:
                v = jnp.where(left_ok, v, pad_val)
            elif dj == 2:
                v = jnp.where(right_ok, v, pad_val)
            isc[t * cin:(t + 1) * cin, :] = v
    return lax.dot_general(
        isc[...], w_ref[...], (((0,), (0,)), ((), ())),
        preferred_element_type=jnp.float32) + b_ref[...]


def _conv3x3(hsc, isc, w_ref, b_ref, x, Wl, HWl, P, pad_val):
    cin = x.shape[1]
    fill = jnp.full((P, cin), pad_val, dtype=jnp.float32)
    hsc[0:P, :] = fill
    hsc[P:P + HWl, :] = x
    hsc[P + HWl:P + HWl + P, :] = fill

    jcol = lax.broadcasted_iota(jnp.int32, (HWl, 1), 0) % Wl
    left_ok = jcol > 0
    right_ok = jcol < Wl - 1

    for di in range(3):
        for dj in range(3):
            t = di * 3 + dj
            off = P + (di - 1) * Wl + (dj - 1)
            v = hsc[pl.ds(off, HWl), :]
            if dj == 0:
                v = jnp.where(left_ok, v, pad_val)
            elif dj == 2:
                v = jnp.where(right_ok, v, pad_val)
            isc[:, t * cin:(t + 1) * cin] = v
    return jnp.dot(isc[...], w_ref[...],
                   preferred_element_type=jnp.float32) + b_ref[...]


def _point_net_kernel(
    x_ref, w1_ref, b1_ref, w2_ref, b2_ref, w3_ref, b3_ref, w4_ref, b4_ref,
    wh_ref, bh_ref,
    score_ref, coord_ref, desc_ref,
    hs1, ic1, hs2, ic2, ds2, hs3, ic3, hs4, ic4, ds4, hsh, ich,
    *, H, W, n_valid,
):
    HW = H * W
    H2, W2 = H // 2, W // 2
    HW2 = H2 * W2
    hc, wc = H // 4, W // 4
    HWc = hc * wc
    P1, P2, P3 = _halo(W), _halo(W2), _halo(wc)
    c2 = w2_ref.shape[1]
    c4 = w4_ref.shape[1]

    x0 = x_ref[0]
    a1 = jnp.maximum(
        _conv3x3_cm_in(hs1, ic1, w1_ref, b1_ref, x0, W, HW, P1, 0.5), 0.0)
    a2 = jnp.maximum(_conv3x3(hs2, ic2, w2_ref, b2_ref, a1, W, HW, P1, 0.0), 0.0)

    ds2[...] = a2.reshape(H2, 2 * W, c2)
    a2d = ds2[:, pl.ds(0, W2, 2), :].reshape(HW2, c2)

    a3 = jnp.maximum(_conv3x3(hs3, ic3, w3_ref, b3_ref, a2d, W2, HW2, P2, 0.0), 0.0)
    a4 = jnp.maximum(_conv3x3(hs4, ic4, w4_ref, b4_ref, a3, W2, HW2, P2, 0.0), 0.0)

    ds4[...] = a4.reshape(H2 // 2, 2 * W2, c4)
    feat = ds4[:, pl.ds(0, wc, 2), :].reshape(HWc, c4)

    h = _conv3x3(hsh, ich, wh_ref, bh_ref, feat, wc, HWc, P3, 0.0)

    col = lax.broadcasted_iota(jnp.int32, h.shape, 1)
    cell = H // hc
    step = (cell - 1) / 2.0
    k = lax.broadcasted_iota(jnp.int32, (HWc, 1), 0)
    bx = (k % wc).astype(jnp.float32) * cell + step
    by = (k // wc).astype(jnp.float32) * cell + step
    base = jnp.where(col == 1, bx, by)
    cmax = jnp.where(col == 1, float(W - 1), float(H - 1))
    coord = jnp.clip(base + jnp.tanh(h) * step, 0.0, cmax)
    score = jax.nn.sigmoid(h)
    out = jnp.where(col == 0, score, jnp.where(col <= 2, coord, h))
    out_t = out.T
    score_ref[0] = out_t[0:1, :]
    coord_ref[0] = out_t[1:3, :]
    desc_ref[0] = out_t[3:n_valid, :]


def kernel(img, w1, b1, w2, b2, w3, b3, w4, b4, ws, bs, wc, bc, wd, bd):
    B, C, H, W = img.shape
    hc, wcell = H // 4, W // 4
    HW, HW2, HWc = H * W, (H // 2) * (W // 2), hc * wcell
    c1 = w1.shape[1]
    c2 = w2.shape[1]
    c3 = w3.shape[1]
    c4 = w4.shape[1]
    c_desc = wd.shape[1]
    n_valid = 3 + c_desc
    NH = 128

    x = img.reshape(B, C, HW)

    inv_std = 1.0 / 0.225
    w1f = w1 * inv_std
    b1f = b1 - (0.5 * inv_std) * jnp.sum(w1, axis=0, keepdims=True)

    wh = jnp.concatenate([ws, wc, wd], axis=1)
    bh = jnp.concatenate([bs, bc, bd], axis=1)
    wh = jnp.pad(wh, ((0, 0), (0, NH - n_valid)))
    bh = jnp.pad(bh, ((0, 0), (0, NH - n_valid)))

    P1, P2, P3 = _halo(W), _halo(W // 2), _halo(wcell)
    full = lambda a: pl.BlockSpec(a.shape, lambda b: (0, 0))
    f32 = jnp.float32

    score, coord, desc = pl.pallas_call(
        functools.partial(_point_net_kernel, H=H, W=W, n_valid=n_valid),
        out_shape=(
            jax.ShapeDtypeStruct((B, 1, HWc), f32),
            jax.ShapeDtypeStruct((B, 2, HWc), f32),
            jax.ShapeDtypeStruct((B, c_desc, HWc), f32),
        ),
        grid=(B,),
        in_specs=[
            pl.BlockSpec((1, C, HW), lambda b: (b, 0, 0)),
            full(w1f), full(b1f),
            full(w2), full(b2),
            full(w3), full(b3),
            full(w4), full(b4),
            full(wh), full(bh),
        ],
        out_specs=(
            pl.BlockSpec((1, 1, HWc), lambda b: (b, 0, 0)),
            pl.BlockSpec((1, 2, HWc), lambda b: (b, 0, 0)),
            pl.BlockSpec((1, c_desc, HWc), lambda b: (b, 0, 0)),
        ),
        scratch_shapes=[
            pltpu.VMEM((C, HW + 2 * P1), f32),
            pltpu.VMEM((9 * C, HW), f32),
            pltpu.VMEM((HW + 2 * P1, c1), f32),
            pltpu.VMEM((HW, 9 * c1), f32),
            pltpu.VMEM((H // 2, 2 * W, c2), f32),
            pltpu.VMEM((HW2 + 2 * P2, c2), f32),
            pltpu.VMEM((HW2, 9 * c2), f32),
            pltpu.VMEM((HW2 + 2 * P2, c3), f32),
            pltpu.VMEM((HW2, 9 * c3), f32),
            pltpu.VMEM((H // 4, 2 * (W // 2), c4), f32),
            pltpu.VMEM((HWc + 2 * P3, c4), f32),
            pltpu.VMEM((HWc, 9 * c4), f32),
        ],
        compiler_params=pltpu.CompilerParams(
            dimension_semantics=("parallel",),
            vmem_limit_bytes=64 * 1024 * 1024,
        ),
    )(x, w1f, b1f, w2, b2, w3, b3, w4, b4, wh, bh)

    return (score.reshape(B, 1, hc, wcell),
            coord.reshape(B, 2, hc, wcell),
            desc.reshape(B, c_desc, hc, wcell))
